# Initial kernel scaffold; baseline (speedup 1.0000x reference)
import jax
import jax.numpy as jnp
from jax.experimental import pallas as pl


def kernel(inputs, embeddings, project_kernel):
    raise NotImplementedError("write your pallas kernel here")


# speedup vs baseline: 5.8123x; 5.8123x over previous
import functools

import jax
import jax.numpy as jnp
from jax import lax
from jax.experimental import pallas as pl
from jax.experimental.pallas import tpu as pltpu
from jax.experimental.pallas import tpu_sc as plsc

_BATCH = 4096
_HIST = 50
_BT = _BATCH * _HIST
_D = 32
_DOUT = 128

_NC = 2
_NS = 16
_NW = _NC * _NS
_BPW = _BT // _NW
_CH = 1600
_NCH = _BPW // _CH


def _sc_gather(idx_flat, table):
    mesh = plsc.VectorSubcoreMesh(core_axis_name="c", subcore_axis_name="s")

    @functools.partial(
        pl.kernel,
        out_type=jax.ShapeDtypeStruct((_BT, _D), jnp.float32),
        mesh=mesh,
        scratch_types=[
            pltpu.VMEM((_CH,), jnp.int32),
            pltpu.VMEM((_CH, _D), jnp.float32),
            pltpu.SemaphoreType.DMA,
        ],
        compiler_params=pltpu.CompilerParams(use_tc_tiling_on_sc=False),
    )
    def gather_kernel(idx_hbm, table_hbm, out_hbm, idx_v, rows_v, sem):
        wid = lax.axis_index("s") * _NC + lax.axis_index("c")
        base = wid * _BPW
        for c in range(_NCH):
            off = base + c * _CH
            pltpu.sync_copy(idx_hbm.at[pl.ds(off, _CH)], idx_v)
            pltpu.async_copy(table_hbm.at[idx_v], rows_v, sem).wait()
            pltpu.sync_copy(rows_v, out_hbm.at[pl.ds(off, _CH)])

    return gather_kernel(idx_flat, table)


def _tc_project(gathered, project_kernel):
    blk = 2048

    def mm_body(g_ref, p_ref, o_ref):
        o_ref[...] = jnp.dot(
            g_ref[...], p_ref[...], preferred_element_type=jnp.float32
        )

    return pl.pallas_call(
        mm_body,
        grid=(_BT // blk,),
        in_specs=[
            pl.BlockSpec((blk, _D), lambda i: (i, 0)),
            pl.BlockSpec((_D, _DOUT), lambda i: (0, 0)),
        ],
        out_specs=pl.BlockSpec((blk, _DOUT), lambda i: (i, 0)),
        out_shape=jax.ShapeDtypeStruct((_BT, _DOUT), jnp.float32),
    )(gathered, project_kernel)


def kernel(inputs, embeddings, project_kernel):
    idx_flat = inputs.astype(jnp.int32).reshape(_BT)
    gathered = _sc_gather(idx_flat, embeddings)
    out = _tc_project(gathered, project_kernel)
    return out.reshape(_BATCH, _HIST, _DOUT)


# speedup vs baseline: 6.8957x; 1.1864x over previous
import functools

import jax
import jax.numpy as jnp
from jax import lax
from jax.experimental import pallas as pl
from jax.experimental.pallas import tpu as pltpu
from jax.experimental.pallas import tpu_sc as plsc

_BATCH = 4096
_HIST = 50
_BT = _BATCH * _HIST
_D = 32
_DOUT = 128

_NC = 2
_NS = 16
_NW = _NC * _NS
_BPW = _BT // _NW
_CH = 1600
_NCH = _BPW // _CH


def _sc_gather(idx_flat, table):
    mesh = plsc.VectorSubcoreMesh(core_axis_name="c", subcore_axis_name="s")

    @functools.partial(
        pl.kernel,
        out_type=jax.ShapeDtypeStruct((_BT, _D), jnp.float32),
        mesh=mesh,
        scratch_types=[
            pltpu.VMEM((_CH,), jnp.int32),
            pltpu.VMEM((_CH, _D), jnp.float32),
            pltpu.SemaphoreType.DMA,
        ],
        compiler_params=pltpu.CompilerParams(use_tc_tiling_on_sc=False),
    )
    def gather_kernel(idx_hbm, table_hbm, out_hbm, idx_v, rows_v, sem):
        wid = lax.axis_index("s") * _NC + lax.axis_index("c")
        base = wid * _BPW
        for c in range(_NCH):
            off = base + c * _CH
            pltpu.sync_copy(idx_hbm.at[pl.ds(off, _CH)], idx_v)
            pltpu.async_copy(table_hbm.at[idx_v], rows_v, sem).wait()
            pltpu.sync_copy(rows_v, out_hbm.at[pl.ds(off, _CH)])

    return gather_kernel(idx_flat, table)


def _tc_project(gathered, project_kernel):
    blk_b = 64

    def mm_body(g_ref, p_ref, o_ref):
        res = jnp.dot(g_ref[...], p_ref[...], preferred_element_type=jnp.float32)
        o_ref[...] = res.reshape(blk_b, _HIST, _DOUT)

    return pl.pallas_call(
        mm_body,
        grid=(_BATCH // blk_b,),
        in_specs=[
            pl.BlockSpec((blk_b * _HIST, _D), lambda i: (i, 0)),
            pl.BlockSpec((_D, _DOUT), lambda i: (0, 0)),
        ],
        out_specs=pl.BlockSpec((blk_b, _HIST, _DOUT), lambda i: (i, 0, 0)),
        out_shape=jax.ShapeDtypeStruct((_BATCH, _HIST, _DOUT), jnp.float32),
    )(gathered, project_kernel)


def kernel(inputs, embeddings, project_kernel):
    idx_flat = inputs.astype(jnp.int32).reshape(_BT)
    gathered = _sc_gather(idx_flat, embeddings)
    return _tc_project(gathered, project_kernel)
